# baseline (speedup 1.0000x reference)
import functools

import jax
import jax.numpy as jnp
from jax import lax
from jax.experimental import pallas as pl
from jax.experimental.pallas import tpu as pltpu
from jax.experimental.pallas import tpu_sc as plsc

_NC = 2
_NS = 16
_NW = _NC * _NS
_C = 80


def _prep_body(x_ref, w_ref, as_ref, ad_ref, hs_ref, asv_ref, adv_ref):
    h = jnp.dot(x_ref[...], w_ref[...], preferred_element_type=jnp.float32)
    d_half = h.shape[1] // 2
    hs_ref[...] = jnp.stack([h[:, :d_half], h[:, d_half:]])
    asv_ref[...] = jnp.sum(h * as_ref[...], axis=1, keepdims=True)
    adv_ref[...] = jnp.sum(h * ad_ref[...], axis=1, keepdims=True)


def _prep(x, w_gat, att_s, att_d):
    n, d_in = x.shape
    d_h = w_gat.shape[1]
    r = 1000
    return pl.pallas_call(
        _prep_body,
        grid=(n // r,),
        in_specs=[
            pl.BlockSpec((r, d_in), lambda i: (i, 0)),
            pl.BlockSpec((d_in, d_h), lambda i: (0, 0)),
            pl.BlockSpec((1, d_h), lambda i: (0, 0)),
            pl.BlockSpec((1, d_h), lambda i: (0, 0)),
        ],
        out_specs=[
            pl.BlockSpec((_NC, r, d_h // 2), lambda i: (0, i, 0)),
            pl.BlockSpec((r, 1), lambda i: (i, 0)),
            pl.BlockSpec((r, 1), lambda i: (i, 0)),
        ],
        out_shape=[
            jax.ShapeDtypeStruct((_NC, n, d_h // 2), jnp.float32),
            jax.ShapeDtypeStruct((n, 1), jnp.float32),
            jax.ShapeDtypeStruct((n, 1), jnp.float32),
        ],
    )(x, w_gat, att_s, att_d)


def _sc_edge_body(n, d_half, nch,
                  src_ref, dst_ref, asrc_ref, adst_ref, hs_ref,
                  num_out, den_out,
                  num_sh, den_sh,
                  asrc_t, adst_t, sidx, didx,
                  gbuf0, dbuf0, wbuf0, rows0,
                  gbuf1, dbuf1, wbuf1, rows1,
                  zden,
                  semg0, semg1, sems0, sems1, semd0, semd1):
    cid = lax.axis_index("c")
    sid = lax.axis_index("s")

    n_rchunk = n // _C
    n_dchunk = n // 400

    pltpu.sync_copy(asrc_ref, asrc_t)
    pltpu.sync_copy(adst_ref, adst_t)
    pltpu.sync_copy(src_ref.at[sid], sidx)
    pltpu.sync_copy(dst_ref.at[sid], didx)

    goff = cid * n

    def scale_rows0(wb):
        @plsc.parallel_loop(0, _C, unroll=8)
        def _(i):
            wv = plsc.load_gather(wb, [jnp.full((16,), i, jnp.int32)])
            for fb in range(d_half // 16):
                sl = pl.ds(16 * fb, 16)
                rows0[i, sl] = rows0[i, sl] * wv

    def init_chunk(k, carry):
        ch = sid + _NS * k
        @pl.when(ch < n_rchunk)
        def _():
            base = _C * ch
            pltpu.sync_copy(hs_ref.at[pl.ds(goff + base, _C)], rows0)
            for j in range(_C // 16):
                sl = pl.ds(16 * j, 16)
                e = asrc_t[pl.ds(base + 16 * j, 16)] \
                    + adst_t[pl.ds(base + 16 * j, 16)]
                e = jnp.where(e > 0, e, jnp.float32(0.2) * e)
                wbuf0[sl] = jnp.exp(e)
            scale_rows0(wbuf0)
            pltpu.sync_copy(rows0, num_sh.at[pl.ds(base, _C), :])
            @pl.when(cid == 0)
            def _():
                pltpu.sync_copy(wbuf0, den_sh.at[pl.ds(base, _C)])
        return carry
    lax.fori_loop(0, pl.cdiv(n_rchunk, _NS), init_chunk, 0)
    plsc.subcore_barrier()

    def compute_w(k, gbuf, dbuf, wbuf):
        for j in range(_C // 16):
            sl = pl.ds(16 * j, 16)
            sv = sidx[k, sl]
            dv = didx[k, sl]
            gbuf[sl] = sv + goff
            dbuf[sl] = dv
            a_s = plsc.load_gather(asrc_t, [sv])
            a_d = plsc.load_gather(adst_t, [dv])
            e = a_s + a_d
            e = jnp.where(e > 0, e, jnp.float32(0.2) * e)
            wbuf[sl] = jnp.exp(e)

    def scale(rows, wbuf):
        @plsc.parallel_loop(0, _C, unroll=8)
        def _(i):
            wv = plsc.load_gather(wbuf, [jnp.full((16,), i, jnp.int32)])
            for fb in range(d_half // 16):
                sl = pl.ds(16 * fb, 16)
                rows[i, sl] = rows[i, sl] * wv

    def start_scatter(rows, wbuf, dbuf, sems, semd):
        pltpu.async_copy(rows, num_sh.at[dbuf], sems, add=True)
        @pl.when(cid == 0)
        def _():
            pltpu.async_copy(wbuf, den_sh.at[dbuf], semd, add=True)

    def wait_scatter(rows, wbuf, dbuf, sems, semd):
        pltpu.make_async_copy(rows, num_sh.at[dbuf], sems).wait()
        @pl.when(cid == 0)
        def _():
            pltpu.make_async_copy(wbuf, den_sh.at[dbuf], semd).wait()

    npair = nch // 2
    compute_w(0, gbuf0, dbuf0, wbuf0)
    pltpu.async_copy(hs_ref.at[gbuf0], rows0, semg0)

    def pair(i, carry):
        k1 = 2 * i + 1
        @pl.when(i > 0)
        def _():
            wait_scatter(rows1, wbuf1, dbuf1, sems1, semd1)
        compute_w(k1, gbuf1, dbuf1, wbuf1)
        pltpu.async_copy(hs_ref.at[gbuf1], rows1, semg1)

        pltpu.make_async_copy(hs_ref.at[gbuf0], rows0, semg0).wait()
        scale(rows0, wbuf0)
        start_scatter(rows0, wbuf0, dbuf0, sems0, semd0)

        pltpu.make_async_copy(hs_ref.at[gbuf1], rows1, semg1).wait()
        scale(rows1, wbuf1)
        start_scatter(rows1, wbuf1, dbuf1, sems1, semd1)

        @pl.when(i < npair - 1)
        def _():
            wait_scatter(rows0, wbuf0, dbuf0, sems0, semd0)
            compute_w(k1 + 1, gbuf0, dbuf0, wbuf0)
            pltpu.async_copy(hs_ref.at[gbuf0], rows0, semg0)
        return carry
    lax.fori_loop(0, npair, pair, 0)
    wait_scatter(rows0, wbuf0, dbuf0, sems0, semd0)
    wait_scatter(rows1, wbuf1, dbuf1, sems1, semd1)
    plsc.subcore_barrier()

    def c_num(k, carry):
        ch = sid + _NS * k
        @pl.when(ch < n_rchunk)
        def _():
            pltpu.sync_copy(num_sh.at[pl.ds(_C * ch, _C), :], rows0)
            pltpu.sync_copy(rows0, num_out.at[cid, pl.ds(_C * ch, _C), :])
        return carry
    lax.fori_loop(0, pl.cdiv(n_rchunk, _NS), c_num, 0)

    @pl.when(cid == 0)
    def _():
        def c_den(k, carry):
            ch = sid + _NS * k
            @pl.when(ch < n_dchunk)
            def _():
                pltpu.sync_copy(den_sh.at[pl.ds(400 * ch, 400)], zden)
                pltpu.sync_copy(zden, den_out.at[pl.ds(400 * ch, 400)])
            return carry
        lax.fori_loop(0, pl.cdiv(n_dchunk, _NS), c_den, 0)


def _sc_edges(src2, dst2, a_src_v, a_dst_v, hs):
    n = a_src_v.shape[0]
    d_half = hs.shape[1]
    nch = src2.shape[1]
    mesh = plsc.VectorSubcoreMesh(core_axis_name="c", subcore_axis_name="s",
                                  num_cores=_NC, num_subcores=_NS)
    body = functools.partial(_sc_edge_body, n, d_half, nch)
    fn = pl.kernel(
        body,
        out_type=(
            jax.ShapeDtypeStruct((_NC, n, d_half), jnp.float32),
            jax.ShapeDtypeStruct((n,), jnp.float32),
        ),
        mesh=mesh,
        compiler_params=pltpu.CompilerParams(needs_layout_passes=False,
                                             use_tc_tiling_on_sc=False),
        scratch_types=[
            pltpu.VMEM_SHARED((n, d_half), jnp.float32),
            pltpu.VMEM_SHARED((n,), jnp.float32),
            pltpu.VMEM((n,), jnp.float32),
            pltpu.VMEM((n,), jnp.float32),
            pltpu.VMEM((nch, _C), jnp.int32),
            pltpu.VMEM((nch, _C), jnp.int32),
            pltpu.VMEM((_C,), jnp.int32),
            pltpu.VMEM((_C,), jnp.int32),
            pltpu.VMEM((_C,), jnp.float32),
            pltpu.VMEM((_C, d_half), jnp.float32),
            pltpu.VMEM((_C,), jnp.int32),
            pltpu.VMEM((_C,), jnp.int32),
            pltpu.VMEM((_C,), jnp.float32),
            pltpu.VMEM((_C, d_half), jnp.float32),
            pltpu.VMEM((400,), jnp.float32),
            pltpu.SemaphoreType.DMA,
            pltpu.SemaphoreType.DMA,
            pltpu.SemaphoreType.DMA,
            pltpu.SemaphoreType.DMA,
            pltpu.SemaphoreType.DMA,
            pltpu.SemaphoreType.DMA,
        ],
    )
    return fn(src2, dst2, a_src_v, a_dst_v, hs)


def _final_body(num_ref, den_ref, ext_ref, bg_ref,
                w1a_ref, w1b_ref, b1_ref, w2_ref, b2_ref, w3_ref, b3_ref,
                out_ref):
    num = jnp.concatenate([num_ref[0], num_ref[1]], axis=1)
    den = den_ref[...]
    g = num / den + bg_ref[...]
    g = jnp.maximum(g, 0.0)
    m = (jnp.dot(g, w1a_ref[...], preferred_element_type=jnp.float32)
         + jnp.dot(ext_ref[...], w1b_ref[...], preferred_element_type=jnp.float32)
         + b1_ref[...])
    m = jnp.maximum(m, 0.0)
    f2 = jnp.dot(m, w2_ref[...], preferred_element_type=jnp.float32) + b2_ref[...]
    out_ref[...] = (jnp.dot(f2, w3_ref[...], preferred_element_type=jnp.float32)
                    + b3_ref[...])


def _final(num, den, ext, bg, w1a, w1b, b1, w2, b2, w3, b3):
    n = ext.shape[0]
    d_h = w2.shape[1]
    d_ext = ext.shape[1]
    d_mlr = w1a.shape[1]
    d_od = w3.shape[1]
    r = 1000
    return pl.pallas_call(
        _final_body,
        grid=(n // r,),
        in_specs=[
            pl.BlockSpec((_NC, r, d_h // 2), lambda i: (0, i, 0)),
            pl.BlockSpec((r, 1), lambda i: (i, 0)),
            pl.BlockSpec((r, d_ext), lambda i: (i, 0)),
            pl.BlockSpec((1, d_h), lambda i: (0, 0)),
            pl.BlockSpec((d_h, d_mlr), lambda i: (0, 0)),
            pl.BlockSpec((d_ext, d_mlr), lambda i: (0, 0)),
            pl.BlockSpec((1, d_mlr), lambda i: (0, 0)),
            pl.BlockSpec((d_mlr, d_h), lambda i: (0, 0)),
            pl.BlockSpec((1, d_h), lambda i: (0, 0)),
            pl.BlockSpec((d_h, d_od), lambda i: (0, 0)),
            pl.BlockSpec((1, d_od), lambda i: (0, 0)),
        ],
        out_specs=pl.BlockSpec((r, d_od), lambda i: (i, 0)),
        out_shape=jax.ShapeDtypeStruct((n, d_od), jnp.float32),
    )(num, den, ext, bg, w1a, w1b, b1, w2, b2, w3, b3)


def kernel(x, edge_index, external, W_gat, att_src, att_dst, b_gat,
           W1, b1, W2, b2, W3, b3):
    n, d_in = x.shape
    d_h = W_gat.shape[1]
    e_tot = edge_index.shape[1]

    d_half = d_h // 2
    hs4, a_s, a_d = _prep(x, W_gat,
                          att_src.reshape(1, d_h),
                          att_dst.reshape(1, d_h))
    hs = hs4.reshape(_NC * n, d_half)

    src2 = edge_index[0].reshape(_NS, e_tot // (_NS * _C), _C)
    dst2 = edge_index[1].reshape(_NS, e_tot // (_NS * _C), _C)

    num, den = _sc_edges(src2, dst2, a_s.reshape(n), a_d.reshape(n), hs)

    return _final(num, den.reshape(n, 1), external,
                  b_gat.reshape(1, d_h), W1[:d_h], W1[d_h:],
                  b1.reshape(1, -1), W2, b2.reshape(1, -1),
                  W3, b3.reshape(1, -1))


# speedup vs baseline: 1.0382x; 1.0382x over previous
import functools

import jax
import jax.numpy as jnp
from jax import lax
from jax.experimental import pallas as pl
from jax.experimental.pallas import tpu as pltpu
from jax.experimental.pallas import tpu_sc as plsc

_NC = 2
_NS = 16
_NW = _NC * _NS
_C = 80


def _prep_body(x_ref, w_ref, as_ref, ad_ref, hs_ref, asv_ref, adv_ref, ws_ref):
    h = jnp.dot(x_ref[...], w_ref[...], preferred_element_type=jnp.float32)
    d_half = h.shape[1] // 2
    hs_ref[...] = jnp.stack([h[:, :d_half], h[:, d_half:]])
    a_s = jnp.sum(h * as_ref[...], axis=1, keepdims=True)
    a_d = jnp.sum(h * ad_ref[...], axis=1, keepdims=True)
    asv_ref[...] = a_s
    adv_ref[...] = a_d
    e = a_s + a_d
    e = jnp.where(e > 0, e, jnp.float32(0.2) * e)
    ws_ref[...] = jnp.exp(e)


def _prep(x, w_gat, att_s, att_d):
    n, d_in = x.shape
    d_h = w_gat.shape[1]
    r = 1000
    return pl.pallas_call(
        _prep_body,
        grid=(n // r,),
        in_specs=[
            pl.BlockSpec((r, d_in), lambda i: (i, 0)),
            pl.BlockSpec((d_in, d_h), lambda i: (0, 0)),
            pl.BlockSpec((1, d_h), lambda i: (0, 0)),
            pl.BlockSpec((1, d_h), lambda i: (0, 0)),
        ],
        out_specs=[
            pl.BlockSpec((_NC, r, d_h // 2), lambda i: (0, i, 0)),
            pl.BlockSpec((r, 1), lambda i: (i, 0)),
            pl.BlockSpec((r, 1), lambda i: (i, 0)),
            pl.BlockSpec((r, 1), lambda i: (i, 0)),
        ],
        out_shape=[
            jax.ShapeDtypeStruct((_NC, n, d_h // 2), jnp.float32),
            jax.ShapeDtypeStruct((n, 1), jnp.float32),
            jax.ShapeDtypeStruct((n, 1), jnp.float32),
            jax.ShapeDtypeStruct((n, 1), jnp.float32),
        ],
    )(x, w_gat, att_s, att_d)


def _sc_edge_body(n, d_half, nch,
                  src_ref, dst_ref, asrc_ref, adst_ref, hs_ref,
                  num_out, den_out,
                  num_sh, den_sh,
                  asrc_t, adst_t, sidx, didx,
                  gbuf0, dbuf0, wbuf0, rows0,
                  gbuf1, dbuf1, wbuf1, rows1,
                  zden,
                  semg0, semg1, sems0, sems1, semd0, semd1):
    cid = lax.axis_index("c")
    sid = lax.axis_index("s")

    n_rchunk = n // _C
    n_dchunk = n // 400

    zero16 = jnp.zeros((16,), jnp.float32)
    n_zj = pl.cdiv(n_rchunk, _NS)

    pltpu.async_copy(asrc_ref, asrc_t, semg0)
    pltpu.async_copy(adst_ref, adst_t, semg1)
    pltpu.async_copy(src_ref.at[sid], sidx, semd0)
    pltpu.async_copy(dst_ref.at[sid], didx, semd1)

    @plsc.parallel_loop(0, _C, unroll=8)
    def _(i):
        for f in range(d_half // 16):
            rows0[i, pl.ds(16 * f, 16)] = zero16
    for i in range(25):
        zden[pl.ds(16 * i, 16)] = zero16

    for j in range(n_zj):
        ch = sid + _NS * j
        @pl.when(ch < n_rchunk)
        def _(ch=ch):
            pltpu.async_copy(rows0, num_sh.at[pl.ds(_C * ch, _C), :], sems0)

    def z_den(k, carry):
        ch = sid + _NS * k
        @pl.when(ch < n_dchunk)
        def _():
            pltpu.sync_copy(zden, den_sh.at[pl.ds(400 * ch, 400)])
        return carry
    lax.fori_loop(0, pl.cdiv(n_dchunk, _NS), z_den, 0)

    pltpu.make_async_copy(asrc_ref, asrc_t, semg0).wait()
    pltpu.make_async_copy(adst_ref, adst_t, semg1).wait()
    pltpu.make_async_copy(src_ref.at[sid], sidx, semd0).wait()
    pltpu.make_async_copy(dst_ref.at[sid], didx, semd1).wait()
    for j in range(n_zj):
        ch = sid + _NS * j
        @pl.when(ch < n_rchunk)
        def _(ch=ch):
            pltpu.make_async_copy(rows0, num_sh.at[pl.ds(_C * ch, _C), :],
                                  sems0).wait()
    plsc.subcore_barrier()

    goff = cid * n

    def compute_w(k, gbuf, dbuf, wbuf):
        for j in range(_C // 16):
            sl = pl.ds(16 * j, 16)
            sv = sidx[k, sl]
            dv = didx[k, sl]
            gbuf[sl] = sv + goff
            dbuf[sl] = dv
            a_s = plsc.load_gather(asrc_t, [sv])
            a_d = plsc.load_gather(adst_t, [dv])
            e = a_s + a_d
            e = jnp.where(e > 0, e, jnp.float32(0.2) * e)
            wbuf[sl] = jnp.exp(e)

    def scale(rows, wbuf):
        @plsc.parallel_loop(0, _C, unroll=8)
        def _(i):
            wv = plsc.load_gather(wbuf, [jnp.full((16,), i, jnp.int32)])
            for fb in range(d_half // 16):
                sl = pl.ds(16 * fb, 16)
                rows[i, sl] = rows[i, sl] * wv

    def start_scatter(rows, wbuf, dbuf, sems, semd):
        pltpu.async_copy(rows, num_sh.at[dbuf], sems, add=True)
        @pl.when(cid == 0)
        def _():
            pltpu.async_copy(wbuf, den_sh.at[dbuf], semd, add=True)

    def wait_scatter(rows, wbuf, dbuf, sems, semd):
        pltpu.make_async_copy(rows, num_sh.at[dbuf], sems).wait()
        @pl.when(cid == 0)
        def _():
            pltpu.make_async_copy(wbuf, den_sh.at[dbuf], semd).wait()

    npair = nch // 2
    compute_w(0, gbuf0, dbuf0, wbuf0)
    pltpu.async_copy(hs_ref.at[gbuf0], rows0, semg0)

    def pair(i, carry):
        k1 = 2 * i + 1
        @pl.when(i > 0)
        def _():
            wait_scatter(rows1, wbuf1, dbuf1, sems1, semd1)
        compute_w(k1, gbuf1, dbuf1, wbuf1)
        pltpu.async_copy(hs_ref.at[gbuf1], rows1, semg1)

        pltpu.make_async_copy(hs_ref.at[gbuf0], rows0, semg0).wait()
        scale(rows0, wbuf0)
        start_scatter(rows0, wbuf0, dbuf0, sems0, semd0)

        pltpu.make_async_copy(hs_ref.at[gbuf1], rows1, semg1).wait()
        scale(rows1, wbuf1)
        start_scatter(rows1, wbuf1, dbuf1, sems1, semd1)

        @pl.when(i < npair - 1)
        def _():
            wait_scatter(rows0, wbuf0, dbuf0, sems0, semd0)
            compute_w(k1 + 1, gbuf0, dbuf0, wbuf0)
            pltpu.async_copy(hs_ref.at[gbuf0], rows0, semg0)
        return carry
    lax.fori_loop(0, npair, pair, 0)
    wait_scatter(rows0, wbuf0, dbuf0, sems0, semd0)
    wait_scatter(rows1, wbuf1, dbuf1, sems1, semd1)
    plsc.subcore_barrier()

    for j in range(n_zj):
        ch = sid + _NS * j
        rbuf = rows0 if j % 2 == 0 else rows1
        ss = sems0 if j % 2 == 0 else sems1
        @pl.when(ch < n_rchunk)
        def _(j=j, ch=ch, rbuf=rbuf, ss=ss):
            if j >= 2:
                pltpu.make_async_copy(
                    rbuf, num_out.at[cid, pl.ds(_C * (ch - 2 * _NS), _C), :],
                    ss).wait()
            pltpu.sync_copy(num_sh.at[pl.ds(_C * ch, _C), :], rbuf)
            pltpu.async_copy(rbuf, num_out.at[cid, pl.ds(_C * ch, _C), :], ss)
    for j in (n_zj - 2, n_zj - 1):
        ch = sid + _NS * j
        rbuf = rows0 if j % 2 == 0 else rows1
        ss = sems0 if j % 2 == 0 else sems1
        @pl.when(ch < n_rchunk)
        def _(ch=ch, rbuf=rbuf, ss=ss):
            pltpu.make_async_copy(
                rbuf, num_out.at[cid, pl.ds(_C * ch, _C), :], ss).wait()

    @pl.when(cid == 0)
    def _():
        def c_den(k, carry):
            ch = sid + _NS * k
            @pl.when(ch < n_dchunk)
            def _():
                pltpu.sync_copy(den_sh.at[pl.ds(400 * ch, 400)], zden)
                pltpu.sync_copy(zden, den_out.at[pl.ds(400 * ch, 400)])
            return carry
        lax.fori_loop(0, pl.cdiv(n_dchunk, _NS), c_den, 0)


def _sc_edges(src2, dst2, a_src_v, a_dst_v, hs):
    n = a_src_v.shape[0]
    d_half = hs.shape[1]
    nch = src2.shape[1]
    mesh = plsc.VectorSubcoreMesh(core_axis_name="c", subcore_axis_name="s",
                                  num_cores=_NC, num_subcores=_NS)
    body = functools.partial(_sc_edge_body, n, d_half, nch)
    fn = pl.kernel(
        body,
        out_type=(
            jax.ShapeDtypeStruct((_NC, n, d_half), jnp.float32),
            jax.ShapeDtypeStruct((n,), jnp.float32),
        ),
        mesh=mesh,
        compiler_params=pltpu.CompilerParams(needs_layout_passes=False,
                                             use_tc_tiling_on_sc=False),
        scratch_types=[
            pltpu.VMEM_SHARED((n, d_half), jnp.float32),
            pltpu.VMEM_SHARED((n,), jnp.float32),
            pltpu.VMEM((n,), jnp.float32),
            pltpu.VMEM((n,), jnp.float32),
            pltpu.VMEM((nch, _C), jnp.int32),
            pltpu.VMEM((nch, _C), jnp.int32),
            pltpu.VMEM((_C,), jnp.int32),
            pltpu.VMEM((_C,), jnp.int32),
            pltpu.VMEM((_C,), jnp.float32),
            pltpu.VMEM((_C, d_half), jnp.float32),
            pltpu.VMEM((_C,), jnp.int32),
            pltpu.VMEM((_C,), jnp.int32),
            pltpu.VMEM((_C,), jnp.float32),
            pltpu.VMEM((_C, d_half), jnp.float32),
            pltpu.VMEM((400,), jnp.float32),
            pltpu.SemaphoreType.DMA,
            pltpu.SemaphoreType.DMA,
            pltpu.SemaphoreType.DMA,
            pltpu.SemaphoreType.DMA,
            pltpu.SemaphoreType.DMA,
            pltpu.SemaphoreType.DMA,
        ],
    )
    return fn(src2, dst2, a_src_v, a_dst_v, hs)


def _final_body(num_ref, den_ref, hs_ref, ext_ref, ws_ref, bg_ref,
                w1a_ref, w1b_ref, b1_ref, w2_ref, b2_ref, w3_ref, b3_ref,
                out_ref):
    h = jnp.concatenate([hs_ref[0], hs_ref[1]], axis=1)
    w_self = ws_ref[...]
    num = jnp.concatenate([num_ref[0], num_ref[1]], axis=1) + w_self * h
    den = den_ref[...] + w_self
    g = num / den + bg_ref[...]
    g = jnp.maximum(g, 0.0)
    m = (jnp.dot(g, w1a_ref[...], preferred_element_type=jnp.float32)
         + jnp.dot(ext_ref[...], w1b_ref[...], preferred_element_type=jnp.float32)
         + b1_ref[...])
    m = jnp.maximum(m, 0.0)
    f2 = jnp.dot(m, w2_ref[...], preferred_element_type=jnp.float32) + b2_ref[...]
    out_ref[...] = (jnp.dot(f2, w3_ref[...], preferred_element_type=jnp.float32)
                    + b3_ref[...])


def _final(num, den, hs4, ext, w_self, bg, w1a, w1b, b1, w2, b2, w3, b3):
    n = ext.shape[0]
    d_h = w2.shape[1]
    d_ext = ext.shape[1]
    d_mlr = w1a.shape[1]
    d_od = w3.shape[1]
    r = 1000
    return pl.pallas_call(
        _final_body,
        grid=(n // r,),
        in_specs=[
            pl.BlockSpec((_NC, r, d_h // 2), lambda i: (0, i, 0)),
            pl.BlockSpec((r, 1), lambda i: (i, 0)),
            pl.BlockSpec((_NC, r, d_h // 2), lambda i: (0, i, 0)),
            pl.BlockSpec((r, d_ext), lambda i: (i, 0)),
            pl.BlockSpec((r, 1), lambda i: (i, 0)),
            pl.BlockSpec((1, d_h), lambda i: (0, 0)),
            pl.BlockSpec((d_h, d_mlr), lambda i: (0, 0)),
            pl.BlockSpec((d_ext, d_mlr), lambda i: (0, 0)),
            pl.BlockSpec((1, d_mlr), lambda i: (0, 0)),
            pl.BlockSpec((d_mlr, d_h), lambda i: (0, 0)),
            pl.BlockSpec((1, d_h), lambda i: (0, 0)),
            pl.BlockSpec((d_h, d_od), lambda i: (0, 0)),
            pl.BlockSpec((1, d_od), lambda i: (0, 0)),
        ],
        out_specs=pl.BlockSpec((r, d_od), lambda i: (i, 0)),
        out_shape=jax.ShapeDtypeStruct((n, d_od), jnp.float32),
    )(num, den, hs4, ext, w_self, bg, w1a, w1b, b1, w2, b2, w3, b3)


def kernel(x, edge_index, external, W_gat, att_src, att_dst, b_gat,
           W1, b1, W2, b2, W3, b3):
    n, d_in = x.shape
    d_h = W_gat.shape[1]
    e_tot = edge_index.shape[1]

    d_half = d_h // 2
    hs4, a_s, a_d, w_self = _prep(x, W_gat,
                                  att_src.reshape(1, d_h),
                                  att_dst.reshape(1, d_h))
    hs = hs4.reshape(_NC * n, d_half)

    src2 = edge_index[0].reshape(_NS, e_tot // (_NS * _C), _C)
    dst2 = edge_index[1].reshape(_NS, e_tot // (_NS * _C), _C)

    num, den = _sc_edges(src2, dst2, a_s.reshape(n), a_d.reshape(n), hs)

    return _final(num, den.reshape(n, 1), hs4, external, w_self,
                  b_gat.reshape(1, d_h), W1[:d_h], W1[d_h:],
                  b1.reshape(1, -1), W2, b2.reshape(1, -1),
                  W3, b3.reshape(1, -1))


# speedup vs baseline: 1.4029x; 1.3513x over previous
import functools

import jax
import jax.numpy as jnp
from jax import lax
from jax.experimental import pallas as pl
from jax.experimental.pallas import tpu as pltpu
from jax.experimental.pallas import tpu_sc as plsc

_NC = 2
_NS = 16
_NW = _NC * _NS
_C = 80


def _prep_body(x_ref, w_ref, as_ref, ad_ref, hs_ref, asv_ref, adv_ref, ws_ref):
    h = jnp.dot(x_ref[...], w_ref[...], preferred_element_type=jnp.float32)
    d_half = h.shape[1] // 2
    hs_ref[...] = jnp.stack([h[:, :d_half], h[:, d_half:]])
    a_s = jnp.sum(h * as_ref[...], axis=1, keepdims=True)
    a_d = jnp.sum(h * ad_ref[...], axis=1, keepdims=True)
    asv_ref[...] = a_s
    adv_ref[...] = a_d
    e = a_s + a_d
    e = jnp.where(e > 0, e, jnp.float32(0.2) * e)
    ws_ref[...] = jnp.exp(e)


def _prep(x, w_gat, att_s, att_d):
    n, d_in = x.shape
    d_h = w_gat.shape[1]
    r = 1000
    return pl.pallas_call(
        _prep_body,
        grid=(n // r,),
        in_specs=[
            pl.BlockSpec((r, d_in), lambda i: (i, 0)),
            pl.BlockSpec((d_in, d_h), lambda i: (0, 0)),
            pl.BlockSpec((1, d_h), lambda i: (0, 0)),
            pl.BlockSpec((1, d_h), lambda i: (0, 0)),
        ],
        out_specs=[
            pl.BlockSpec((_NC, r, d_h // 2), lambda i: (0, i, 0)),
            pl.BlockSpec((r, 1), lambda i: (i, 0)),
            pl.BlockSpec((r, 1), lambda i: (i, 0)),
            pl.BlockSpec((r, 1), lambda i: (i, 0)),
        ],
        out_shape=[
            jax.ShapeDtypeStruct((_NC, n, d_h // 2), jnp.float32),
            jax.ShapeDtypeStruct((n, 1), jnp.float32),
            jax.ShapeDtypeStruct((n, 1), jnp.float32),
            jax.ShapeDtypeStruct((n, 1), jnp.float32),
        ],
    )(x, w_gat, att_s, att_d)


def _sc_edge_body(n, d_half, nch,
                  src_ref, dst_ref, asrc_ref, adst_ref, hs_ref,
                  num_out, den_out,
                  num_sh, den_sh,
                  asrc_t, adst_t, sidx, didx,
                  gbuf0, dbuf0, wbuf0, rows0,
                  gbuf1, dbuf1, wbuf1, rows1,
                  gbuf2, dbuf2, wbuf2, rows2,
                  gbuf3, dbuf3, wbuf3, rows3,
                  zden,
                  semg0, semg1, sems0, sems1, semd0, semd1,
                  semg2, semg3, sems2, sems3, semd2, semd3):
    cid = lax.axis_index("c")
    sid = lax.axis_index("s")

    n_rchunk = n // _C
    n_dchunk = n // 400

    zero16 = jnp.zeros((16,), jnp.float32)
    n_zj = pl.cdiv(n_rchunk, _NS)

    pltpu.async_copy(asrc_ref, asrc_t, semg0)
    pltpu.async_copy(adst_ref, adst_t, semg1)
    pltpu.async_copy(src_ref.at[sid], sidx, semd0)
    pltpu.async_copy(dst_ref.at[sid], didx, semd1)

    @plsc.parallel_loop(0, _C, unroll=8)
    def _(i):
        for f in range(d_half // 16):
            rows0[i, pl.ds(16 * f, 16)] = zero16
    for i in range(25):
        zden[pl.ds(16 * i, 16)] = zero16

    for j in range(n_zj):
        ch = sid + _NS * j
        @pl.when(ch < n_rchunk)
        def _(ch=ch):
            pltpu.async_copy(rows0, num_sh.at[pl.ds(_C * ch, _C), :], sems0)

    def z_den(k, carry):
        ch = sid + _NS * k
        @pl.when(ch < n_dchunk)
        def _():
            pltpu.sync_copy(zden, den_sh.at[pl.ds(400 * ch, 400)])
        return carry
    lax.fori_loop(0, pl.cdiv(n_dchunk, _NS), z_den, 0)

    pltpu.make_async_copy(asrc_ref, asrc_t, semg0).wait()
    pltpu.make_async_copy(adst_ref, adst_t, semg1).wait()
    pltpu.make_async_copy(src_ref.at[sid], sidx, semd0).wait()
    pltpu.make_async_copy(dst_ref.at[sid], didx, semd1).wait()
    for j in range(n_zj):
        ch = sid + _NS * j
        @pl.when(ch < n_rchunk)
        def _(ch=ch):
            pltpu.make_async_copy(rows0, num_sh.at[pl.ds(_C * ch, _C), :],
                                  sems0).wait()
    plsc.subcore_barrier()

    goff = cid * n

    def compute_w(k, gbuf, dbuf, wbuf):
        for j in range(_C // 16):
            sl = pl.ds(16 * j, 16)
            sv = sidx[k, sl]
            dv = didx[k, sl]
            gbuf[sl] = sv + goff
            dbuf[sl] = dv
            a_s = plsc.load_gather(asrc_t, [sv])
            a_d = plsc.load_gather(adst_t, [dv])
            e = a_s + a_d
            e = jnp.where(e > 0, e, jnp.float32(0.2) * e)
            wbuf[sl] = jnp.exp(e)

    def scale(rows, wbuf):
        @plsc.parallel_loop(0, _C, unroll=8)
        def _(i):
            wv = plsc.load_gather(wbuf, [jnp.full((16,), i, jnp.int32)])
            for fb in range(d_half // 16):
                sl = pl.ds(16 * fb, 16)
                rows[i, sl] = rows[i, sl] * wv

    def start_scatter(rows, wbuf, dbuf, sems, semd):
        pltpu.async_copy(rows, num_sh.at[dbuf], sems, add=True)
        @pl.when(cid == 0)
        def _():
            pltpu.async_copy(wbuf, den_sh.at[dbuf], semd, add=True)

    def wait_scatter(rows, wbuf, dbuf, sems, semd):
        pltpu.make_async_copy(rows, num_sh.at[dbuf], sems).wait()
        @pl.when(cid == 0)
        def _():
            pltpu.make_async_copy(wbuf, den_sh.at[dbuf], semd).wait()

    sets = [
        (gbuf0, dbuf0, wbuf0, rows0, semg0, sems0, semd0),
        (gbuf1, dbuf1, wbuf1, rows1, semg1, sems1, semd1),
        (gbuf2, dbuf2, wbuf2, rows2, semg2, sems2, semd2),
        (gbuf3, dbuf3, wbuf3, rows3, semg3, sems3, semd3),
    ]

    def prep(k, s_):
        g, d, w, r, sg, _, _ = s_
        compute_w(k, g, d, w)
        pltpu.async_copy(hs_ref.at[g], r, sg)

    def wait_set(s_):
        g, d, w, r, _, ss, sd = s_
        wait_scatter(r, w, d, ss, sd)

    def proc(k, s_):
        g, d, w, r, sg, ss, sd = s_
        pltpu.make_async_copy(hs_ref.at[g], r, sg).wait()
        scale(r, w)
        start_scatter(r, w, d, ss, sd)

    nquad = (nch - 2) // 4
    assert nquad * 4 + 2 == nch
    prep(0, sets[0])
    prep(1, sets[1])

    def quad(i, carry):
        for s in range(4):
            k = 4 * i + s
            sp = sets[(s + 2) % 4]
            if s < 2:
                @pl.when(i > 0)
                def _(sp=sp):
                    wait_set(sp)
            else:
                wait_set(sp)
            prep(k + 2, sp)
            proc(k, sets[s])
        return carry
    lax.fori_loop(0, nquad, quad, 0)
    proc(nch - 2, sets[0])
    proc(nch - 1, sets[1])
    wait_set(sets[2])
    wait_set(sets[3])
    wait_set(sets[0])
    wait_set(sets[1])
    plsc.subcore_barrier()

    for j in range(n_zj):
        ch = sid + _NS * j
        rbuf = rows0 if j % 2 == 0 else rows1
        ss = sems0 if j % 2 == 0 else sems1
        @pl.when(ch < n_rchunk)
        def _(j=j, ch=ch, rbuf=rbuf, ss=ss):
            if j >= 2:
                pltpu.make_async_copy(
                    rbuf, num_out.at[cid, pl.ds(_C * (ch - 2 * _NS), _C), :],
                    ss).wait()
            pltpu.sync_copy(num_sh.at[pl.ds(_C * ch, _C), :], rbuf)
            pltpu.async_copy(rbuf, num_out.at[cid, pl.ds(_C * ch, _C), :], ss)
    for j in (n_zj - 2, n_zj - 1):
        ch = sid + _NS * j
        rbuf = rows0 if j % 2 == 0 else rows1
        ss = sems0 if j % 2 == 0 else sems1
        @pl.when(ch < n_rchunk)
        def _(ch=ch, rbuf=rbuf, ss=ss):
            pltpu.make_async_copy(
                rbuf, num_out.at[cid, pl.ds(_C * ch, _C), :], ss).wait()

    @pl.when(cid == 0)
    def _():
        def c_den(k, carry):
            ch = sid + _NS * k
            @pl.when(ch < n_dchunk)
            def _():
                pltpu.sync_copy(den_sh.at[pl.ds(400 * ch, 400)], zden)
                pltpu.sync_copy(zden, den_out.at[pl.ds(400 * ch, 400)])
            return carry
        lax.fori_loop(0, pl.cdiv(n_dchunk, _NS), c_den, 0)


def _sc_edges(src2, dst2, a_src_v, a_dst_v, hs):
    n = a_src_v.shape[0]
    d_half = hs.shape[1]
    nch = src2.shape[1]
    mesh = plsc.VectorSubcoreMesh(core_axis_name="c", subcore_axis_name="s",
                                  num_cores=_NC, num_subcores=_NS)
    body = functools.partial(_sc_edge_body, n, d_half, nch)
    fn = pl.kernel(
        body,
        out_type=(
            jax.ShapeDtypeStruct((_NC, n, d_half), jnp.float32),
            jax.ShapeDtypeStruct((n,), jnp.float32),
        ),
        mesh=mesh,
        compiler_params=pltpu.CompilerParams(needs_layout_passes=False,
                                             use_tc_tiling_on_sc=False),
        scratch_types=[
            pltpu.VMEM_SHARED((n, d_half), jnp.float32),
            pltpu.VMEM_SHARED((n,), jnp.float32),
            pltpu.VMEM((n,), jnp.float32),
            pltpu.VMEM((n,), jnp.float32),
            pltpu.VMEM((nch, _C), jnp.int32),
            pltpu.VMEM((nch, _C), jnp.int32),
            pltpu.VMEM((_C,), jnp.int32),
            pltpu.VMEM((_C,), jnp.int32),
            pltpu.VMEM((_C,), jnp.float32),
            pltpu.VMEM((_C, d_half), jnp.float32),
            pltpu.VMEM((_C,), jnp.int32),
            pltpu.VMEM((_C,), jnp.int32),
            pltpu.VMEM((_C,), jnp.float32),
            pltpu.VMEM((_C, d_half), jnp.float32),
            pltpu.VMEM((_C,), jnp.int32),
            pltpu.VMEM((_C,), jnp.int32),
            pltpu.VMEM((_C,), jnp.float32),
            pltpu.VMEM((_C, d_half), jnp.float32),
            pltpu.VMEM((_C,), jnp.int32),
            pltpu.VMEM((_C,), jnp.int32),
            pltpu.VMEM((_C,), jnp.float32),
            pltpu.VMEM((_C, d_half), jnp.float32),
            pltpu.VMEM((400,), jnp.float32),
        ] + [pltpu.SemaphoreType.DMA] * 12,
    )
    return fn(src2, dst2, a_src_v, a_dst_v, hs)


def _final_body(num_ref, den_ref, hs_ref, ext_ref, ws_ref, bg_ref,
                w1a_ref, w1b_ref, b1_ref, w2_ref, b2_ref, w3_ref, b3_ref,
                out_ref):
    h = jnp.concatenate([hs_ref[0], hs_ref[1]], axis=1)
    w_self = ws_ref[...]
    num = jnp.concatenate([num_ref[0], num_ref[1]], axis=1) + w_self * h
    den = den_ref[...] + w_self
    g = num / den + bg_ref[...]
    g = jnp.maximum(g, 0.0)
    m = (jnp.dot(g, w1a_ref[...], preferred_element_type=jnp.float32)
         + jnp.dot(ext_ref[...], w1b_ref[...], preferred_element_type=jnp.float32)
         + b1_ref[...])
    m = jnp.maximum(m, 0.0)
    f2 = jnp.dot(m, w2_ref[...], preferred_element_type=jnp.float32) + b2_ref[...]
    out_ref[...] = (jnp.dot(f2, w3_ref[...], preferred_element_type=jnp.float32)
                    + b3_ref[...])


def _final(num, den, hs4, ext, w_self, bg, w1a, w1b, b1, w2, b2, w3, b3):
    n = ext.shape[0]
    d_h = w2.shape[1]
    d_ext = ext.shape[1]
    d_mlr = w1a.shape[1]
    d_od = w3.shape[1]
    r = 1000
    return pl.pallas_call(
        _final_body,
        grid=(n // r,),
        in_specs=[
            pl.BlockSpec((_NC, r, d_h // 2), lambda i: (0, i, 0)),
            pl.BlockSpec((r, 1), lambda i: (i, 0)),
            pl.BlockSpec((_NC, r, d_h // 2), lambda i: (0, i, 0)),
            pl.BlockSpec((r, d_ext), lambda i: (i, 0)),
            pl.BlockSpec((r, 1), lambda i: (i, 0)),
            pl.BlockSpec((1, d_h), lambda i: (0, 0)),
            pl.BlockSpec((d_h, d_mlr), lambda i: (0, 0)),
            pl.BlockSpec((d_ext, d_mlr), lambda i: (0, 0)),
            pl.BlockSpec((1, d_mlr), lambda i: (0, 0)),
            pl.BlockSpec((d_mlr, d_h), lambda i: (0, 0)),
            pl.BlockSpec((1, d_h), lambda i: (0, 0)),
            pl.BlockSpec((d_h, d_od), lambda i: (0, 0)),
            pl.BlockSpec((1, d_od), lambda i: (0, 0)),
        ],
        out_specs=pl.BlockSpec((r, d_od), lambda i: (i, 0)),
        out_shape=jax.ShapeDtypeStruct((n, d_od), jnp.float32),
    )(num, den, hs4, ext, w_self, bg, w1a, w1b, b1, w2, b2, w3, b3)


def kernel(x, edge_index, external, W_gat, att_src, att_dst, b_gat,
           W1, b1, W2, b2, W3, b3):
    n, d_in = x.shape
    d_h = W_gat.shape[1]
    e_tot = edge_index.shape[1]

    d_half = d_h // 2
    hs4, a_s, a_d, w_self = _prep(x, W_gat,
                                  att_src.reshape(1, d_h),
                                  att_dst.reshape(1, d_h))
    hs = hs4.reshape(_NC * n, d_half)

    src2 = edge_index[0].reshape(_NS, e_tot // (_NS * _C), _C)
    dst2 = edge_index[1].reshape(_NS, e_tot // (_NS * _C), _C)

    num, den = _sc_edges(src2, dst2, a_s.reshape(n), a_d.reshape(n), hs)

    return _final(num, den.reshape(n, 1), hs4, external, w_self,
                  b_gat.reshape(1, d_h), W1[:d_h], W1[d_h:],
                  b1.reshape(1, -1), W2, b2.reshape(1, -1),
                  W3, b3.reshape(1, -1))
